# Initial kernel scaffold; baseline (speedup 1.0000x reference)
import jax
import jax.numpy as jnp
from jax.experimental import pallas as pl
from jax.experimental.pallas import tpu as pltpu
from jax.experimental.pallas import tpu_sc as plsc

_NC = 2
_NS = 16
_K = 400
_DEGW = 16


def _sc_mesh():
    return plsc.VectorSubcoreMesh(core_axis_name="c", subcore_axis_name="s")


def _make_deg_kernel(n, e):
    epc = e // _NC
    epw = epc // _NS
    rpw = n // _NS

    @pl.kernel(
        out_type=jax.ShapeDtypeStruct((_NC, n, _DEGW), jnp.float32),
        mesh=_sc_mesh(),
        scratch_types=[
            pltpu.VMEM_SHARED((n, _DEGW), jnp.float32),
            pltpu.VMEM((1, _K), jnp.int32),
            pltpu.VMEM((_K, _DEGW), jnp.float32),
        ],
    )
    def deg_kernel(ei_hbm, zeros_hbm, ones_hbm, out_hbm, acc, dbuf, ones_v):
        c = jax.lax.axis_index("c")
        s = jax.lax.axis_index("s")
        pltpu.sync_copy(zeros_hbm.at[pl.ds(s * rpw, rpw)],
                        acc.at[pl.ds(s * rpw, rpw)])
        pltpu.sync_copy(ones_hbm, ones_v)
        plsc.subcore_barrier()

        base = c * epc + s * epw

        @pl.loop(0, epw, step=_K)
        def _(k):
            pltpu.sync_copy(ei_hbm.at[pl.ds(1, 1), pl.ds(base + k, _K)], dbuf)
            pltpu.sync_copy(ones_v, acc.at[dbuf.at[0]], add=True)

        plsc.subcore_barrier()
        pltpu.sync_copy(acc.at[pl.ds(s * rpw, rpw)],
                        out_hbm.at[c].at[pl.ds(s * rpw, rpw)])

    return deg_kernel


def _make_agg_kernel(n, e, h):
    epc = e // _NC
    epw = epc // _NS
    rpw = n // _NS

    @pl.kernel(
        out_type=jax.ShapeDtypeStruct((_NC, n, h), jnp.float32),
        mesh=_sc_mesh(),
        scratch_types=[
            pltpu.VMEM_SHARED((n, h), jnp.float32),
            pltpu.VMEM((1, _K), jnp.int32),
            pltpu.VMEM((1, _K), jnp.int32),
            pltpu.VMEM((_K, h), jnp.float32),
        ],
    )
    def agg_kernel(y_hbm, ei_hbm, zeros_hbm, out_hbm, acc, sbuf, dbuf, rows):
        c = jax.lax.axis_index("c")
        s = jax.lax.axis_index("s")
        pltpu.sync_copy(zeros_hbm.at[pl.ds(s * rpw, rpw)],
                        acc.at[pl.ds(s * rpw, rpw)])
        plsc.subcore_barrier()

        base = c * epc + s * epw

        @pl.loop(0, epw, step=_K)
        def _(k):
            pltpu.sync_copy(ei_hbm.at[pl.ds(0, 1), pl.ds(base + k, _K)], sbuf)
            pltpu.sync_copy(ei_hbm.at[pl.ds(1, 1), pl.ds(base + k, _K)], dbuf)
            pltpu.sync_copy(y_hbm.at[sbuf.at[0]], rows)
            pltpu.sync_copy(rows, acc.at[dbuf.at[0]], add=True)

        plsc.subcore_barrier()
        pltpu.sync_copy(acc.at[pl.ds(s * rpw, rpw)],
                        out_hbm.at[c].at[pl.ds(s * rpw, rpw)])

    return agg_kernel


_B = 1000


def _stage0_body(d_ref, x_ref, w_ref, y_ref, dinv_ref):
    deg = d_ref[0, :, 0:1] + d_ref[1, :, 0:1] + 1.0
    dinv = jax.lax.rsqrt(deg)
    xw = jnp.dot(x_ref[...], w_ref[...], preferred_element_type=jnp.float32)
    y_ref[...] = xw * dinv
    dinv_ref[...] = jnp.broadcast_to(dinv, y_ref.shape)


def _mid_body(p_ref, y_ref, dinv_ref, b_ref, g_ref, be_ref, w_ref, yout_ref):
    agg = p_ref[0] + p_ref[1] + y_ref[...]
    t = agg * dinv_ref[...] + b_ref[...]
    t = jnp.maximum(t, 0.0)
    mu = jnp.mean(t, axis=1, keepdims=True)
    var = jnp.mean((t - mu) ** 2, axis=1, keepdims=True)
    t = (t - mu) * jax.lax.rsqrt(var + 1e-5) * g_ref[...] + be_ref[...]
    yout_ref[...] = jnp.dot(t, w_ref[...],
                            preferred_element_type=jnp.float32) * dinv_ref[...]


def _head_body(p_ref, y_ref, dinv_ref, b_ref, wp0_ref, bp0_ref, wp1_ref,
               bp1_ref, out_ref):
    hcur = p_ref[0] + p_ref[1] + y_ref[...]
    hcur = jnp.maximum(hcur * dinv_ref[...] + b_ref[...], 0.0)
    t = jnp.dot(hcur, wp0_ref[...],
                preferred_element_type=jnp.float32) + bp0_ref[...]
    out_ref[...] = jnp.dot(t, wp1_ref[...],
                           preferred_element_type=jnp.float32) + bp1_ref[...]


def _row_spec(b, cols):
    return pl.BlockSpec((b, cols), lambda i: (i, 0))


def _full_spec(shape):
    return pl.BlockSpec(shape, lambda i: tuple(0 for _ in shape))


def kernel(x, edge_index, W0, b0, W1, b1, W2, b2, g0, be0, g1, be1,
           Wp0, bp0, Wp1, bp1):
    n, d = x.shape
    e = edge_index.shape[1]
    h = W0.shape[1]
    o = Wp1.shape[1]
    grid = (n // _B,)

    zeros_nh = jnp.zeros((n, h), jnp.float32)
    zeros_nd = jnp.zeros((n, _DEGW), jnp.float32)
    ones_k = jnp.ones((_K, _DEGW), jnp.float32)

    deg_parts = _make_deg_kernel(n, e)(edge_index, zeros_nd, ones_k)
    agg = _make_agg_kernel(n, e, h)

    b0r, b1r, b2r = (v.reshape(1, h) for v in (b0, b1, b2))
    g0r, g1r = g0.reshape(1, h), g1.reshape(1, h)
    be0r, be1r = be0.reshape(1, h), be1.reshape(1, h)
    bp0r, bp1r = bp0.reshape(1, h), bp1.reshape(1, o)

    y0, dinv = pl.pallas_call(
        _stage0_body,
        grid=grid,
        in_specs=[
            pl.BlockSpec((_NC, _B, _DEGW), lambda i: (0, i, 0)),
            _row_spec(_B, d),
            _full_spec((d, h)),
        ],
        out_specs=[_row_spec(_B, h), _row_spec(_B, h)],
        out_shape=[
            jax.ShapeDtypeStruct((n, h), jnp.float32),
            jax.ShapeDtypeStruct((n, h), jnp.float32),
        ],
    )(deg_parts, x, W0)

    def mid(parts, yprev, br, gr, ber, wnext):
        return pl.pallas_call(
            _mid_body,
            grid=grid,
            in_specs=[
                pl.BlockSpec((_NC, _B, h), lambda i: (0, i, 0)),
                _row_spec(_B, h),
                _row_spec(_B, h),
                _full_spec((1, h)),
                _full_spec((1, h)),
                _full_spec((1, h)),
                _full_spec((h, h)),
            ],
            out_specs=_row_spec(_B, h),
            out_shape=jax.ShapeDtypeStruct((n, h), jnp.float32),
        )(parts, yprev, dinv, br, gr, ber, wnext)

    p0 = agg(y0, edge_index, zeros_nh)
    y1 = mid(p0, y0, b0r, g0r, be0r, W1)
    p1 = agg(y1, edge_index, zeros_nh)
    y2 = mid(p1, y1, b1r, g1r, be1r, W2)
    p2 = agg(y2, edge_index, zeros_nh)

    out = pl.pallas_call(
        _head_body,
        grid=grid,
        in_specs=[
            pl.BlockSpec((_NC, _B, h), lambda i: (0, i, 0)),
            _row_spec(_B, h),
            _row_spec(_B, h),
            _full_spec((1, h)),
            _full_spec((h, h)),
            _full_spec((1, h)),
            _full_spec((h, o)),
            _full_spec((1, o)),
        ],
        out_specs=_row_spec(_B, o),
        out_shape=jax.ShapeDtypeStruct((n, o), jnp.float32),
    )(p2, y2, dinv, b2r, Wp0, bp0r, Wp1, bp1r)

    return out


# speedup vs baseline: 16.5114x; 16.5114x over previous
import jax
import jax.numpy as jnp
from jax.experimental import pallas as pl
from jax.experimental.pallas import tpu as pltpu
from jax.experimental.pallas import tpu_sc as plsc

_NC = 2
_NS = 16
_K = 200
_DEGW = 16


def _sc_mesh():
    return plsc.VectorSubcoreMesh(core_axis_name="c", subcore_axis_name="s")


_RB = 1000


def _make_deg_kernel(n, e):
    epc = e // _NC
    epw = epc // _NS
    nrw = n // _RB

    @pl.kernel(
        out_type=jax.ShapeDtypeStruct((_NC, n, _DEGW), jnp.float32),
        mesh=_sc_mesh(),
        scratch_types=[
            pltpu.VMEM_SHARED((n, _DEGW), jnp.float32),
            pltpu.VMEM((_K,), jnp.int32),
            pltpu.VMEM((_K, _DEGW), jnp.float32),
        ],
    )
    def deg_kernel(dst_hbm, zeros_hbm, ones_hbm, out_hbm, acc, dbuf, ones_v):
        c = jax.lax.axis_index("c")
        s = jax.lax.axis_index("s")
        @pl.when(s < nrw)
        def _():
            pltpu.sync_copy(zeros_hbm.at[pl.ds(s * _RB, _RB)],
                            acc.at[pl.ds(s * _RB, _RB)])
        pltpu.sync_copy(ones_hbm, ones_v)
        plsc.subcore_barrier()

        base = c * epc + s * epw

        @pl.loop(0, epw, step=_K)
        def _(k):
            pltpu.sync_copy(dst_hbm.at[pl.ds(base + k, _K)], dbuf)
            pltpu.sync_copy(ones_v, acc.at[dbuf], add=True)

        plsc.subcore_barrier()

        @pl.when(s < nrw)
        def _():
            pltpu.sync_copy(acc.at[pl.ds(s * _RB, _RB)],
                            out_hbm.at[c].at[pl.ds(s * _RB, _RB)])

    return deg_kernel


def _make_agg_kernel(n, e, h):
    epc = e // _NC
    epw = epc // _NS
    nrw = n // _RB

    @pl.kernel(
        out_type=jax.ShapeDtypeStruct((_NC, n, h), jnp.float32),
        mesh=_sc_mesh(),
        scratch_types=[
            pltpu.VMEM_SHARED((n, h), jnp.float32),
            pltpu.VMEM((_K,), jnp.int32),
            pltpu.VMEM((_K,), jnp.int32),
            pltpu.VMEM((_K, h), jnp.float32),
        ],
    )
    def agg_kernel(y_hbm, src_hbm, dst_hbm, zeros_hbm, out_hbm, acc, sbuf, dbuf, rows):
        c = jax.lax.axis_index("c")
        s = jax.lax.axis_index("s")

        @pl.when(s < nrw)
        def _():
            pltpu.sync_copy(zeros_hbm.at[pl.ds(s * _RB, _RB)],
                            acc.at[pl.ds(s * _RB, _RB)])

        plsc.subcore_barrier()

        base = c * epc + s * epw

        @pl.loop(0, epw, step=_K)
        def _(k):
            pltpu.sync_copy(src_hbm.at[pl.ds(base + k, _K)], sbuf)
            pltpu.sync_copy(dst_hbm.at[pl.ds(base + k, _K)], dbuf)
            pltpu.sync_copy(y_hbm.at[sbuf], rows)
            pltpu.sync_copy(rows, acc.at[dbuf], add=True)

        plsc.subcore_barrier()

        @pl.when(s < nrw)
        def _():
            pltpu.sync_copy(acc.at[pl.ds(s * _RB, _RB)],
                            out_hbm.at[c].at[pl.ds(s * _RB, _RB)])

    return agg_kernel


_B = 1000


def _stage0_body(d_ref, x_ref, w_ref, y_ref, dinv_ref):
    deg = d_ref[0, :, 0:1] + d_ref[1, :, 0:1] + 1.0
    dinv = jax.lax.rsqrt(deg)
    xw = jnp.dot(x_ref[...], w_ref[...], preferred_element_type=jnp.float32)
    y_ref[...] = xw * dinv
    dinv_ref[...] = jnp.broadcast_to(dinv, y_ref.shape)


def _mid_body(p_ref, y_ref, dinv_ref, b_ref, g_ref, be_ref, w_ref, yout_ref):
    agg = p_ref[0] + p_ref[1] + y_ref[...]
    t = agg * dinv_ref[...] + b_ref[...]
    t = jnp.maximum(t, 0.0)
    mu = jnp.mean(t, axis=1, keepdims=True)
    var = jnp.mean((t - mu) ** 2, axis=1, keepdims=True)
    t = (t - mu) * jax.lax.rsqrt(var + 1e-5) * g_ref[...] + be_ref[...]
    yout_ref[...] = jnp.dot(t, w_ref[...],
                            preferred_element_type=jnp.float32) * dinv_ref[...]


def _head_body(p_ref, y_ref, dinv_ref, b_ref, wp0_ref, bp0_ref, wp1_ref,
               bp1_ref, out_ref):
    hcur = p_ref[0] + p_ref[1] + y_ref[...]
    hcur = jnp.maximum(hcur * dinv_ref[...] + b_ref[...], 0.0)
    t = jnp.dot(hcur, wp0_ref[...],
                preferred_element_type=jnp.float32) + bp0_ref[...]
    out_ref[...] = jnp.dot(t, wp1_ref[...],
                           preferred_element_type=jnp.float32) + bp1_ref[...]


def _row_spec(b, cols):
    return pl.BlockSpec((b, cols), lambda i: (i, 0))


def _full_spec(shape):
    return pl.BlockSpec(shape, lambda i: tuple(0 for _ in shape))


def kernel(x, edge_index, W0, b0, W1, b1, W2, b2, g0, be0, g1, be1,
           Wp0, bp0, Wp1, bp1):
    n, d = x.shape
    e = edge_index.shape[1]
    h = W0.shape[1]
    o = Wp1.shape[1]
    grid = (n // _B,)

    zeros_nh = jnp.zeros((n, h), jnp.float32)
    zeros_nd = jnp.zeros((n, _DEGW), jnp.float32)
    ones_k = jnp.ones((_K, _DEGW), jnp.float32)

    src_idx = edge_index[0]
    dst_idx = edge_index[1]
    deg_parts = _make_deg_kernel(n, e)(dst_idx, zeros_nd, ones_k)
    agg = _make_agg_kernel(n, e, h)

    b0r, b1r, b2r = (v.reshape(1, h) for v in (b0, b1, b2))
    g0r, g1r = g0.reshape(1, h), g1.reshape(1, h)
    be0r, be1r = be0.reshape(1, h), be1.reshape(1, h)
    bp0r, bp1r = bp0.reshape(1, h), bp1.reshape(1, o)

    y0, dinv = pl.pallas_call(
        _stage0_body,
        grid=grid,
        in_specs=[
            pl.BlockSpec((_NC, _B, _DEGW), lambda i: (0, i, 0)),
            _row_spec(_B, d),
            _full_spec((d, h)),
        ],
        out_specs=[_row_spec(_B, h), _row_spec(_B, h)],
        out_shape=[
            jax.ShapeDtypeStruct((n, h), jnp.float32),
            jax.ShapeDtypeStruct((n, h), jnp.float32),
        ],
    )(deg_parts, x, W0)

    def mid(parts, yprev, br, gr, ber, wnext):
        return pl.pallas_call(
            _mid_body,
            grid=grid,
            in_specs=[
                pl.BlockSpec((_NC, _B, h), lambda i: (0, i, 0)),
                _row_spec(_B, h),
                _row_spec(_B, h),
                _full_spec((1, h)),
                _full_spec((1, h)),
                _full_spec((1, h)),
                _full_spec((h, h)),
            ],
            out_specs=_row_spec(_B, h),
            out_shape=jax.ShapeDtypeStruct((n, h), jnp.float32),
        )(parts, yprev, dinv, br, gr, ber, wnext)

    p0 = agg(y0, src_idx, dst_idx, zeros_nh)
    y1 = mid(p0, y0, b0r, g0r, be0r, W1)
    p1 = agg(y1, src_idx, dst_idx, zeros_nh)
    y2 = mid(p1, y1, b1r, g1r, be1r, W2)
    p2 = agg(y2, src_idx, dst_idx, zeros_nh)

    out = pl.pallas_call(
        _head_body,
        grid=grid,
        in_specs=[
            pl.BlockSpec((_NC, _B, h), lambda i: (0, i, 0)),
            _row_spec(_B, h),
            _row_spec(_B, h),
            _full_spec((1, h)),
            _full_spec((h, h)),
            _full_spec((1, h)),
            _full_spec((h, o)),
            _full_spec((1, o)),
        ],
        out_specs=_row_spec(_B, o),
        out_shape=jax.ShapeDtypeStruct((n, o), jnp.float32),
    )(p2, y2, dinv, b2r, Wp0, bp0r, Wp1, bp1r)

    return out


# speedup vs baseline: 20.4207x; 1.2368x over previous
import jax
import jax.numpy as jnp
from jax.experimental import pallas as pl
from jax.experimental.pallas import tpu as pltpu
from jax.experimental.pallas import tpu_sc as plsc

_NC = 2
_NS = 16
_K = 176
_DEGW = 16
_RB = 1000
_TR = 8


def _sc_mesh():
    return plsc.VectorSubcoreMesh(core_axis_name="c", subcore_axis_name="s")


def _make_deg_kernel(n, epw):
    nch = epw // _K
    n4 = nch - (nch % 4)
    nrw = n // _RB

    @pl.kernel(
        out_type=jax.ShapeDtypeStruct((_NC, n, _DEGW), jnp.float32),
        mesh=_sc_mesh(),
        scratch_types=[
            pltpu.VMEM_SHARED((n + _TR, _DEGW), jnp.float32),
            pltpu.VMEM((_K, _DEGW), jnp.float32),
        ] + [pltpu.VMEM((_K,), jnp.int32) for _ in range(4)]
          + [pltpu.SemaphoreType.DMA for _ in range(5)],
    )
    def deg_kernel(dst_hbm, zeros_hbm, ones_hbm, out_hbm, acc, ones_v,
                   d0, d1, d2, d3, sd0, sd1, sd2, sd3, ss):
        c = jax.lax.axis_index("c")
        s = jax.lax.axis_index("s")
        base = (c * _NS + s) * epw
        dbufs = ((d0, sd0), (d1, sd1), (d2, sd2), (d3, sd3))

        @pl.when(s < nrw)
        def _():
            pltpu.sync_copy(zeros_hbm.at[pl.ds(s * _RB, _RB)],
                            acc.at[pl.ds(s * _RB, _RB)])

        pltpu.sync_copy(ones_hbm, ones_v)
        for t in range(4):
            if t < nch:
                pltpu.async_copy(dst_hbm.at[pl.ds(base + t * _K, _K)],
                                 dbufs[t][0], dbufs[t][1])
        plsc.subcore_barrier()

        @pl.loop(0, n4, step=4)
        def _(g):
            for b in range(4):
                dbuf, sd = dbufs[b]
                pltpu.make_async_copy(dst_hbm.at[pl.ds(base, _K)], dbuf,
                                      sd).wait()
                pltpu.sync_copy(ones_v, acc.at[dbuf], add=True)

                @pl.when(g + 4 + b < nch)
                def _():
                    pltpu.async_copy(
                        dst_hbm.at[pl.ds(base + (g + 4 + b) * _K, _K)],
                        dbuf, sd)

        for r in range(nch % 4):
            dbuf, sd = dbufs[r]
            pltpu.make_async_copy(dst_hbm.at[pl.ds(base, _K)], dbuf,
                                  sd).wait()
            pltpu.sync_copy(ones_v, acc.at[dbuf], add=True)

        plsc.subcore_barrier()

        @pl.when(s < nrw)
        def _():
            pltpu.sync_copy(acc.at[pl.ds(s * _RB, _RB)],
                            out_hbm.at[c].at[pl.ds(s * _RB, _RB)])

    return deg_kernel


def _make_agg_kernel(n, epw, h):
    nch = epw // _K
    n4 = nch - (nch % 4)
    nrw = n // _RB

    @pl.kernel(
        out_type=jax.ShapeDtypeStruct((_NC, n, h), jnp.float32),
        mesh=_sc_mesh(),
        scratch_types=[
            pltpu.VMEM_SHARED((n + _TR, h), jnp.float32),
            pltpu.VMEM((_K, h), jnp.float32),
            pltpu.VMEM((_K, h), jnp.float32),
        ] + [pltpu.VMEM((_K,), jnp.int32) for _ in range(8)]
          + [pltpu.SemaphoreType.DMA for _ in range(10)],
    )
    def agg_kernel(y_hbm, src_hbm, dst_hbm, zeros_hbm, out_hbm, acc,
                   r0, r1, i0, i1, i2, i3, d0, d1, d2, d3,
                   sg0, sg1, si0, si1, si2, si3, sdm0, sdm1, sdm2, sdm3):
        c = jax.lax.axis_index("c")
        s = jax.lax.axis_index("s")
        base = (c * _NS + s) * epw
        rbufs = ((r0, sg0), (r1, sg1))
        ibufs = ((i0, si0), (i1, si1), (i2, si2), (i3, si3))
        dbufs = ((d0, sdm0), (d1, sdm1), (d2, sdm2), (d3, sdm3))

        @pl.when(s < nrw)
        def _():
            pltpu.sync_copy(zeros_hbm.at[pl.ds(s * _RB, _RB)],
                            acc.at[pl.ds(s * _RB, _RB)])

        for t in range(4):
            pltpu.async_copy(src_hbm.at[pl.ds(base + t * _K, _K)],
                             ibufs[t][0], ibufs[t][1])
            pltpu.async_copy(dst_hbm.at[pl.ds(base + t * _K, _K)],
                             dbufs[t][0], dbufs[t][1])
        plsc.subcore_barrier()

        _PIPELINE = False
        if _PIPELINE:
            for t in range(2):
                ibuf, si = ibufs[t]
                pltpu.make_async_copy(src_hbm.at[pl.ds(base, _K)], ibuf,
                                      si).wait()
                pltpu.async_copy(y_hbm.at[ibuf], rbufs[t][0], rbufs[t][1])

        def step_seq(j, q, b):
            rbuf, sg = rbufs[b]
            ibuf, si = ibufs[q]
            dbuf, sd = dbufs[q]
            pltpu.make_async_copy(src_hbm.at[pl.ds(base, _K)], ibuf,
                                  si).wait()
            pltpu.make_async_copy(dst_hbm.at[pl.ds(base, _K)], dbuf,
                                  sd).wait()
            pltpu.sync_copy(y_hbm.at[ibuf], rbuf)
            pltpu.sync_copy(rbuf, acc.at[dbuf], add=True)

            @pl.when(j + 4 < nch)
            def _():
                pltpu.async_copy(src_hbm.at[pl.ds(base + (j + 4) * _K, _K)],
                                 ibuf, si)
                pltpu.async_copy(dst_hbm.at[pl.ds(base + (j + 4) * _K, _K)],
                                 dbuf, sd)

        def step(j, q, b):
            rbuf, sg = rbufs[b]
            ibuf, si = ibufs[q]
            dbuf, sd = dbufs[q]
            pltpu.make_async_copy(y_hbm.at[ibuf], rbuf, sg).wait()
            pltpu.make_async_copy(dst_hbm.at[pl.ds(base, _K)], dbuf,
                                  sd).wait()
            pltpu.sync_copy(rbuf, acc.at[dbuf], add=True)
            @pl.when(j + 4 < nch)
            def _():
                pltpu.async_copy(src_hbm.at[pl.ds(base + (j + 4) * _K, _K)],
                                 ibuf, si)
                pltpu.async_copy(dst_hbm.at[pl.ds(base + (j + 4) * _K, _K)],
                                 dbuf, sd)
            @pl.when(j + 2 < nch)
            def _():
                nibuf, nsi = ibufs[(q + 2) % 4]
                pltpu.make_async_copy(src_hbm.at[pl.ds(base, _K)], nibuf,
                                      nsi).wait()
                pltpu.async_copy(y_hbm.at[nibuf], rbuf, sg)

        _step = step if _PIPELINE else step_seq

        @pl.loop(0, n4, step=4)
        def _(i):
            for b in range(4):
                _step(i + b, b, b % 2)

        for r in range(nch % 4):
            _step(n4 + r, r, r % 2)

        plsc.subcore_barrier()

        @pl.when(s < nrw)
        def _():
            pltpu.sync_copy(acc.at[pl.ds(s * _RB, _RB)],
                            out_hbm.at[c].at[pl.ds(s * _RB, _RB)])

    return agg_kernel


_B = 1000


def _stage0_body(d_ref, x_ref, w_ref, y_ref, dinv_ref):
    deg = d_ref[0, :, 0:1] + d_ref[1, :, 0:1] + 1.0
    dinv = jax.lax.rsqrt(deg)
    xw = jnp.dot(x_ref[...], w_ref[...], preferred_element_type=jnp.float32)
    y_ref[...] = xw * dinv
    dinv_ref[...] = jnp.broadcast_to(dinv, y_ref.shape)


def _mid_body(p_ref, y_ref, dinv_ref, b_ref, g_ref, be_ref, w_ref, yout_ref):
    agg = p_ref[0] + p_ref[1] + y_ref[...]
    t = agg * dinv_ref[...] + b_ref[...]
    t = jnp.maximum(t, 0.0)
    mu = jnp.mean(t, axis=1, keepdims=True)
    var = jnp.mean((t - mu) ** 2, axis=1, keepdims=True)
    t = (t - mu) * jax.lax.rsqrt(var + 1e-5) * g_ref[...] + be_ref[...]
    yout_ref[...] = jnp.dot(t, w_ref[...],
                            preferred_element_type=jnp.float32) * dinv_ref[...]


def _head_body(p_ref, y_ref, dinv_ref, b_ref, wp0_ref, bp0_ref, wp1_ref,
               bp1_ref, out_ref):
    hcur = p_ref[0] + p_ref[1] + y_ref[...]
    hcur = jnp.maximum(hcur * dinv_ref[...] + b_ref[...], 0.0)
    t = jnp.dot(hcur, wp0_ref[...],
                preferred_element_type=jnp.float32) + bp0_ref[...]
    out_ref[...] = jnp.dot(t, wp1_ref[...],
                           preferred_element_type=jnp.float32) + bp1_ref[...]


def _row_spec(b, cols):
    return pl.BlockSpec((b, cols), lambda i: (i, 0))


def _full_spec(shape):
    return pl.BlockSpec(shape, lambda i: tuple(0 for _ in shape))


def kernel(x, edge_index, W0, b0, W1, b1, W2, b2, g0, be0, g1, be1,
           Wp0, bp0, Wp1, bp1):
    n, d = x.shape
    e = edge_index.shape[1]
    h = W0.shape[1]
    o = Wp1.shape[1]
    grid = (n // _B,)
    nw = _NC * _NS

    epw = -(-e // (nw * _K)) * _K
    pad = nw * epw - e
    pad_ar = jnp.arange(pad, dtype=jnp.int32)
    src_p = jnp.concatenate([edge_index[0], pad_ar % n])
    dst_p = jnp.concatenate([edge_index[1], n + (pad_ar % _TR)])

    zeros_nh = jnp.zeros((n, h), jnp.float32)
    zeros_nd = jnp.zeros((n, _DEGW), jnp.float32)
    ones_k = jnp.ones((_K, _DEGW), jnp.float32)

    deg_parts = _make_deg_kernel(n, epw)(dst_p, zeros_nd, ones_k)
    agg = _make_agg_kernel(n, epw, h)

    b0r, b1r, b2r = (v.reshape(1, h) for v in (b0, b1, b2))
    g0r, g1r = g0.reshape(1, h), g1.reshape(1, h)
    be0r, be1r = be0.reshape(1, h), be1.reshape(1, h)
    bp0r, bp1r = bp0.reshape(1, h), bp1.reshape(1, o)

    y0, dinv = pl.pallas_call(
        _stage0_body,
        grid=grid,
        in_specs=[
            pl.BlockSpec((_NC, _B, _DEGW), lambda i: (0, i, 0)),
            _row_spec(_B, d),
            _full_spec((d, h)),
        ],
        out_specs=[_row_spec(_B, h), _row_spec(_B, h)],
        out_shape=[
            jax.ShapeDtypeStruct((n, h), jnp.float32),
            jax.ShapeDtypeStruct((n, h), jnp.float32),
        ],
    )(deg_parts, x, W0)

    def mid(parts, yprev, br, gr, ber, wnext):
        return pl.pallas_call(
            _mid_body,
            grid=grid,
            in_specs=[
                pl.BlockSpec((_NC, _B, h), lambda i: (0, i, 0)),
                _row_spec(_B, h),
                _row_spec(_B, h),
                _full_spec((1, h)),
                _full_spec((1, h)),
                _full_spec((1, h)),
                _full_spec((h, h)),
            ],
            out_specs=_row_spec(_B, h),
            out_shape=jax.ShapeDtypeStruct((n, h), jnp.float32),
        )(parts, yprev, dinv, br, gr, ber, wnext)

    p0 = agg(y0, src_p, dst_p, zeros_nh)
    y1 = mid(p0, y0, b0r, g0r, be0r, W1)
    p1 = agg(y1, src_p, dst_p, zeros_nh)
    y2 = mid(p1, y1, b1r, g1r, be1r, W2)
    p2 = agg(y2, src_p, dst_p, zeros_nh)

    out = pl.pallas_call(
        _head_body,
        grid=grid,
        in_specs=[
            pl.BlockSpec((_NC, _B, h), lambda i: (0, i, 0)),
            _row_spec(_B, h),
            _row_spec(_B, h),
            _full_spec((1, h)),
            _full_spec((h, h)),
            _full_spec((1, h)),
            _full_spec((h, o)),
            _full_spec((1, o)),
        ],
        out_specs=_row_spec(_B, o),
        out_shape=jax.ShapeDtypeStruct((n, o), jnp.float32),
    )(p2, y2, dinv, b2r, Wp0, bp0r, Wp1, bp1r)

    return out


# speedup vs baseline: 21.7035x; 1.0628x over previous
import jax
import jax.numpy as jnp
from jax.experimental import pallas as pl
from jax.experimental.pallas import tpu as pltpu
from jax.experimental.pallas import tpu_sc as plsc

_NC = 2
_NS = 16
_K = 320
_DEGW = 16
_RB = 1000
_TR = 8


def _sc_mesh():
    return plsc.VectorSubcoreMesh(core_axis_name="c", subcore_axis_name="s")


def _make_deg_kernel(n, epw):
    nch = epw // _K
    n4 = nch - (nch % 4)
    nrw = n // _RB

    @pl.kernel(
        out_type=jax.ShapeDtypeStruct((_NC, n, _DEGW), jnp.float32),
        mesh=_sc_mesh(),
        scratch_types=[
            pltpu.VMEM_SHARED((n + _TR, _DEGW), jnp.float32),
            pltpu.VMEM((_K, _DEGW), jnp.float32),
        ] + [pltpu.VMEM((_K,), jnp.int32) for _ in range(4)]
          + [pltpu.SemaphoreType.DMA for _ in range(5)],
    )
    def deg_kernel(dst_hbm, zeros_hbm, ones_hbm, out_hbm, acc, ones_v,
                   d0, d1, d2, d3, sd0, sd1, sd2, sd3, ss):
        c = jax.lax.axis_index("c")
        s = jax.lax.axis_index("s")
        base = (c * _NS + s) * epw
        dbufs = ((d0, sd0), (d1, sd1), (d2, sd2), (d3, sd3))

        @pl.when(s < nrw)
        def _():
            pltpu.sync_copy(zeros_hbm.at[pl.ds(s * _RB, _RB)],
                            acc.at[pl.ds(s * _RB, _RB)])

        pltpu.sync_copy(ones_hbm, ones_v)
        for t in range(4):
            if t < nch:
                pltpu.async_copy(dst_hbm.at[pl.ds(base + t * _K, _K)],
                                 dbufs[t][0], dbufs[t][1])
        plsc.subcore_barrier()

        @pl.loop(0, n4, step=4)
        def _(g):
            for b in range(4):
                dbuf, sd = dbufs[b]
                pltpu.make_async_copy(dst_hbm.at[pl.ds(base, _K)], dbuf,
                                      sd).wait()
                pltpu.sync_copy(ones_v, acc.at[dbuf], add=True)

                @pl.when(g + 4 + b < nch)
                def _():
                    pltpu.async_copy(
                        dst_hbm.at[pl.ds(base + (g + 4 + b) * _K, _K)],
                        dbuf, sd)

        for r in range(nch % 4):
            dbuf, sd = dbufs[r]
            pltpu.make_async_copy(dst_hbm.at[pl.ds(base, _K)], dbuf,
                                  sd).wait()
            pltpu.sync_copy(ones_v, acc.at[dbuf], add=True)

        plsc.subcore_barrier()

        @pl.when(s < nrw)
        def _():
            pltpu.sync_copy(acc.at[pl.ds(s * _RB, _RB)],
                            out_hbm.at[c].at[pl.ds(s * _RB, _RB)])

    return deg_kernel


def _make_agg_kernel(n, epw, h):
    nch = epw // _K
    n4 = nch - (nch % 4)
    nrw = n // _RB

    @pl.kernel(
        out_type=jax.ShapeDtypeStruct((_NC, n, h), jnp.float32),
        mesh=_sc_mesh(),
        scratch_types=[
            pltpu.VMEM_SHARED((n + _TR, h), jnp.float32),
            pltpu.VMEM((_K, h), jnp.float32),
        ] + [pltpu.VMEM((_K,), jnp.int32) for _ in range(8)]
          + [pltpu.SemaphoreType.DMA for _ in range(8)],
    )
    def agg_kernel(y_hbm, src_hbm, dst_hbm, zeros_hbm, out_hbm, acc,
                   r0, i0, i1, i2, i3, d0, d1, d2, d3,
                   si0, si1, si2, si3, sdm0, sdm1, sdm2, sdm3):
        c = jax.lax.axis_index("c")
        s = jax.lax.axis_index("s")
        base = (c * _NS + s) * epw
        ibufs = ((i0, si0), (i1, si1), (i2, si2), (i3, si3))
        dbufs = ((d0, sdm0), (d1, sdm1), (d2, sdm2), (d3, sdm3))

        @pl.when(s < nrw)
        def _():
            pltpu.sync_copy(zeros_hbm.at[pl.ds(s * _RB, _RB)],
                            acc.at[pl.ds(s * _RB, _RB)])

        for t in range(4):
            pltpu.async_copy(src_hbm.at[pl.ds(base + t * _K, _K)],
                             ibufs[t][0], ibufs[t][1])
            pltpu.async_copy(dst_hbm.at[pl.ds(base + t * _K, _K)],
                             dbufs[t][0], dbufs[t][1])
        plsc.subcore_barrier()

        def step(j, q, b):
            rbuf = r0
            ibuf, si = ibufs[q]
            dbuf, sd = dbufs[q]
            pltpu.make_async_copy(src_hbm.at[pl.ds(base, _K)], ibuf,
                                  si).wait()
            pltpu.make_async_copy(dst_hbm.at[pl.ds(base, _K)], dbuf,
                                  sd).wait()
            pltpu.sync_copy(y_hbm.at[ibuf], rbuf)
            pltpu.sync_copy(rbuf, acc.at[dbuf], add=True)

            @pl.when(j + 4 < nch)
            def _():
                pltpu.async_copy(src_hbm.at[pl.ds(base + (j + 4) * _K, _K)],
                                 ibuf, si)
                pltpu.async_copy(dst_hbm.at[pl.ds(base + (j + 4) * _K, _K)],
                                 dbuf, sd)

        @pl.loop(0, n4, step=4)
        def _(i):
            for b in range(4):
                step(i + b, b, b % 2)

        for r in range(nch % 4):
            step(n4 + r, r, r % 2)

        plsc.subcore_barrier()

        @pl.when(s < nrw)
        def _():
            pltpu.sync_copy(acc.at[pl.ds(s * _RB, _RB)],
                            out_hbm.at[c].at[pl.ds(s * _RB, _RB)])

    return agg_kernel


_B = 1000


def _stage0_body(d_ref, x_ref, w_ref, y_ref, dinv_ref):
    deg = d_ref[0, :, 0:1] + d_ref[1, :, 0:1] + 1.0
    dinv = jax.lax.rsqrt(deg)
    xw = jnp.dot(x_ref[...], w_ref[...], preferred_element_type=jnp.float32)
    y_ref[...] = xw * dinv
    dinv_ref[...] = jnp.broadcast_to(dinv, y_ref.shape)


def _mid_body(p_ref, y_ref, dinv_ref, b_ref, g_ref, be_ref, w_ref, yout_ref):
    agg = p_ref[0] + p_ref[1] + y_ref[...]
    t = agg * dinv_ref[...] + b_ref[...]
    t = jnp.maximum(t, 0.0)
    mu = jnp.mean(t, axis=1, keepdims=True)
    var = jnp.mean((t - mu) ** 2, axis=1, keepdims=True)
    t = (t - mu) * jax.lax.rsqrt(var + 1e-5) * g_ref[...] + be_ref[...]
    yout_ref[...] = jnp.dot(t, w_ref[...],
                            preferred_element_type=jnp.float32) * dinv_ref[...]


def _head_body(p_ref, y_ref, dinv_ref, b_ref, wp0_ref, bp0_ref, wp1_ref,
               bp1_ref, out_ref):
    hcur = p_ref[0] + p_ref[1] + y_ref[...]
    hcur = jnp.maximum(hcur * dinv_ref[...] + b_ref[...], 0.0)
    t = jnp.dot(hcur, wp0_ref[...],
                preferred_element_type=jnp.float32) + bp0_ref[...]
    out_ref[...] = jnp.dot(t, wp1_ref[...],
                           preferred_element_type=jnp.float32) + bp1_ref[...]


def _row_spec(b, cols):
    return pl.BlockSpec((b, cols), lambda i: (i, 0))


def _full_spec(shape):
    return pl.BlockSpec(shape, lambda i: tuple(0 for _ in shape))


def kernel(x, edge_index, W0, b0, W1, b1, W2, b2, g0, be0, g1, be1,
           Wp0, bp0, Wp1, bp1):
    n, d = x.shape
    e = edge_index.shape[1]
    h = W0.shape[1]
    o = Wp1.shape[1]
    grid = (n // _B,)
    nw = _NC * _NS

    epw = -(-e // (nw * _K)) * _K
    pad = nw * epw - e
    pad_ar = jnp.arange(pad, dtype=jnp.int32)
    src_p = jnp.concatenate([edge_index[0], pad_ar % n])
    dst_p = jnp.concatenate([edge_index[1], n + (pad_ar % _TR)])

    zeros_nh = jnp.zeros((n, h), jnp.float32)
    zeros_nd = jnp.zeros((n, _DEGW), jnp.float32)
    ones_k = jnp.ones((_K, _DEGW), jnp.float32)

    deg_parts = _make_deg_kernel(n, epw)(dst_p, zeros_nd, ones_k)
    agg = _make_agg_kernel(n, epw, h)

    b0r, b1r, b2r = (v.reshape(1, h) for v in (b0, b1, b2))
    g0r, g1r = g0.reshape(1, h), g1.reshape(1, h)
    be0r, be1r = be0.reshape(1, h), be1.reshape(1, h)
    bp0r, bp1r = bp0.reshape(1, h), bp1.reshape(1, o)

    y0, dinv = pl.pallas_call(
        _stage0_body,
        grid=grid,
        in_specs=[
            pl.BlockSpec((_NC, _B, _DEGW), lambda i: (0, i, 0)),
            _row_spec(_B, d),
            _full_spec((d, h)),
        ],
        out_specs=[_row_spec(_B, h), _row_spec(_B, h)],
        out_shape=[
            jax.ShapeDtypeStruct((n, h), jnp.float32),
            jax.ShapeDtypeStruct((n, h), jnp.float32),
        ],
    )(deg_parts, x, W0)

    def mid(parts, yprev, br, gr, ber, wnext):
        return pl.pallas_call(
            _mid_body,
            grid=grid,
            in_specs=[
                pl.BlockSpec((_NC, _B, h), lambda i: (0, i, 0)),
                _row_spec(_B, h),
                _row_spec(_B, h),
                _full_spec((1, h)),
                _full_spec((1, h)),
                _full_spec((1, h)),
                _full_spec((h, h)),
            ],
            out_specs=_row_spec(_B, h),
            out_shape=jax.ShapeDtypeStruct((n, h), jnp.float32),
        )(parts, yprev, dinv, br, gr, ber, wnext)

    p0 = agg(y0, src_p, dst_p, zeros_nh)
    y1 = mid(p0, y0, b0r, g0r, be0r, W1)
    p1 = agg(y1, src_p, dst_p, zeros_nh)
    y2 = mid(p1, y1, b1r, g1r, be1r, W2)
    p2 = agg(y2, src_p, dst_p, zeros_nh)

    out = pl.pallas_call(
        _head_body,
        grid=grid,
        in_specs=[
            pl.BlockSpec((_NC, _B, h), lambda i: (0, i, 0)),
            _row_spec(_B, h),
            _row_spec(_B, h),
            _full_spec((1, h)),
            _full_spec((h, h)),
            _full_spec((1, h)),
            _full_spec((h, o)),
            _full_spec((1, o)),
        ],
        out_specs=_row_spec(_B, o),
        out_shape=jax.ShapeDtypeStruct((n, o), jnp.float32),
    )(p2, y2, dinv, b2r, Wp0, bp0r, Wp1, bp1r)

    return out
